# baseline (speedup 1.0000x reference)
import functools

import jax
import jax.numpy as jnp
from jax import lax
from jax.experimental import pallas as pl
from jax.experimental.pallas import tpu as pltpu
from jax.experimental.pallas import tpu_sc as plsc

NC = 2
NS = 16
L = 16
NW = NC * NS

B = 16
A = 262144
TOTAL = B * A
PER_W = TOTAL // NW
CHUNK = 16384
NCHUNK = PER_W // CHUNK
UNROLL = 8
STEPS = CHUNK // (L * UNROLL)

_LOG1P_C = (
    -0.01720779923130605,
    0.081725645293587,
    -0.1887808235518326,
    0.31458917399063396,
    -0.49697743071943007,
    0.9997923620654827,
    3.511021356138903e-06,
)


def _masked_ce(m, l0, l1):
    d = l1 - l0
    t = jnp.where(m == 1.0, -d, d)
    a = jnp.abs(t)
    u = jnp.exp(-a)
    p = jnp.full((L,), _LOG1P_C[0], jnp.float32)
    for c in _LOG1P_C[1:]:
        p = p * u + c
    ce = jnp.maximum(t, 0.0) + p
    w = m != 0.0
    return jnp.where(w, ce, 0.0), jnp.where(w, 1.0, 0.0)


def _sc_body(match_hbm, logits_hbm, out_sum, out_cnt,
             mb0, mb1, lb0, lb1, acc_s, cnt_s, sm0, sm1, sl0, sl1):
    cid = lax.axis_index("c")
    sid = lax.axis_index("s")
    wid = sid * NC + cid
    mbase = wid * PER_W

    mbufs = (mb0, mb1)
    lbufs = (lb0, lb1)
    msems = (sm0, sm1)
    lsems = (sl0, sl1)

    iota = jnp.arange(L, dtype=jnp.int32)

    def start(k):
        slot = k % 2
        off = mbase + k * CHUNK
        cm = pltpu.async_copy(match_hbm.at[pl.ds(off, CHUNK)],
                              mbufs[slot], msems[slot])
        cl = pltpu.async_copy(logits_hbm.at[pl.ds(2 * off, 2 * CHUNK)],
                              lbufs[slot], lsems[slot])
        return cm, cl

    pending = start(0)
    acc = jnp.zeros((L,), jnp.float32)
    cnt = jnp.zeros((L,), jnp.float32)
    for k in range(NCHUNK):
        nxt = start(k + 1) if k + 1 < NCHUNK else None
        pending[0].wait()
        pending[1].wait()
        mb = mbufs[k % 2]
        lb = lbufs[k % 2]

        def body(j, carry, mb=mb, lb=lb):
            acc, cnt = carry
            for uu in range(UNROLL):
                o = (j * UNROLL + uu) * L
                m = mb[pl.ds(o, L)]
                i0 = (o + iota) * 2
                l0 = plsc.load_gather(lb, [i0])
                l1 = plsc.load_gather(lb, [i0 + 1])
                ce, w = _masked_ce(m, l0, l1)
                acc = acc + ce
                cnt = cnt + w
            return acc, cnt

        acc, cnt = lax.fori_loop(0, STEPS, body, (acc, cnt))
        pending = nxt

    acc_s[...] = acc
    cnt_s[...] = cnt
    pltpu.sync_copy(acc_s, out_sum.at[wid])
    pltpu.sync_copy(cnt_s, out_cnt.at[wid])


_rpn_loss_sc = functools.partial(
    pl.kernel,
    out_type=(jax.ShapeDtypeStruct((NW, L), jnp.float32),
              jax.ShapeDtypeStruct((NW, L), jnp.float32)),
    mesh=plsc.VectorSubcoreMesh(core_axis_name="c", subcore_axis_name="s",
                                num_cores=NC, num_subcores=NS),
    compiler_params=pltpu.CompilerParams(needs_layout_passes=False),
    scratch_types=[
        pltpu.VMEM((CHUNK,), jnp.float32),
        pltpu.VMEM((CHUNK,), jnp.float32),
        pltpu.VMEM((2 * CHUNK,), jnp.float32),
        pltpu.VMEM((2 * CHUNK,), jnp.float32),
        pltpu.VMEM((L,), jnp.float32),
        pltpu.VMEM((L,), jnp.float32),
        pltpu.SemaphoreType.DMA,
        pltpu.SemaphoreType.DMA,
        pltpu.SemaphoreType.DMA,
        pltpu.SemaphoreType.DMA,
    ],
)(_sc_body)


def kernel(rpn_match, rpn_class_logits):
    m = rpn_match.reshape(TOTAL)
    lg = rpn_class_logits.reshape(TOTAL * 2)
    sums, cnts = _rpn_loss_sc(m, lg)
    s = jnp.sum(sums)
    c = jnp.sum(cnts)
    return jnp.where(c > 0, s / jnp.maximum(c, 1.0), jnp.float32(0.0))


# speedup vs baseline: 86.2431x; 86.2431x over previous
import functools

import jax
import jax.numpy as jnp
from jax import lax
from jax.experimental import pallas as pl
from jax.experimental.pallas import tpu as pltpu
from jax.experimental.pallas import tpu_sc as plsc

NC = 2
NS = 16
L = 16
NW = NC * NS

B = 16
A = 262144
TOTAL = B * A
PER_W = TOTAL // NW
CHUNK = 16384
NCHUNK = PER_W // CHUNK
GROUP = 128
STEPS = CHUNK // GROUP
UNROLL = GROUP // L

_LOG1P_C = (
    -0.05486231128935009,
    0.2164085836818178,
    -0.46407070110262433,
    0.9954266617754363,
    0.00014158017492720682,
)


def _sc_body(match_hbm, logits_hbm, out_sum, out_cnt,
             mb0, mb1, lb0, lb1, acc_s, cnt_s, sm0, sm1, sl0, sl1):
    cid = lax.axis_index("c")
    sid = lax.axis_index("s")
    wid = sid * NC + cid
    mbase = wid * PER_W

    mbufs = (mb0, mb1)
    lbufs = (lb0, lb1)
    msems = (sm0, sm1)
    lsems = (sl0, sl1)

    def start(k):
        slot = k % 2
        off = mbase + k * CHUNK
        cm = pltpu.async_copy(match_hbm.at[pl.ds(off, CHUNK)],
                              mbufs[slot], msems[slot])
        cl = pltpu.async_copy(logits_hbm.at[pl.ds(2 * off, 2 * CHUNK)],
                              lbufs[slot], lsems[slot])
        return cm, cl

    pending = start(0)
    acc = jnp.zeros((L,), jnp.float32)
    cnt = jnp.zeros((L,), jnp.float32)
    for k in range(NCHUNK):
        nxt = start(k + 1) if k + 1 < NCHUNK else None
        pending[0].wait()
        pending[1].wait()
        mb = mbufs[k % 2]
        lb = lbufs[k % 2]

        def body(j, carry, mb=mb, lb=lb):
            acc, cnt = carry
            mo = j * GROUP
            lo = j * (2 * GROUP)
            for u in range(UNROLL):
                m = mb[pl.ds(mo + u * L, L)]
                l0 = lb[pl.ds(lo + u * L, L)]
                l1 = lb[pl.ds(lo + GROUP + u * L, L)]
                d = l1 - l0
                md = m * d
                w2 = m * m
                a = jnp.abs(d)
                e = jnp.exp(-a)
                p = jnp.full((L,), _LOG1P_C[0], jnp.float32)
                for c in _LOG1P_C[1:]:
                    p = p * e + c
                ce = p - jnp.minimum(md, 0.0)
                acc = acc + w2 * ce
                cnt = cnt + w2
            return acc, cnt

        acc, cnt = lax.fori_loop(0, STEPS, body, (acc, cnt))
        pending = nxt

    acc_s[...] = acc
    cnt_s[...] = cnt
    pltpu.sync_copy(acc_s, out_sum.at[wid])
    pltpu.sync_copy(cnt_s, out_cnt.at[wid])


_rpn_loss_sc = functools.partial(
    pl.kernel,
    out_type=(jax.ShapeDtypeStruct((NW, L), jnp.float32),
              jax.ShapeDtypeStruct((NW, L), jnp.float32)),
    mesh=plsc.VectorSubcoreMesh(core_axis_name="c", subcore_axis_name="s",
                                num_cores=NC, num_subcores=NS),
    compiler_params=pltpu.CompilerParams(needs_layout_passes=False),
    scratch_types=[
        pltpu.VMEM((CHUNK,), jnp.float32),
        pltpu.VMEM((CHUNK,), jnp.float32),
        pltpu.VMEM((2 * CHUNK,), jnp.float32),
        pltpu.VMEM((2 * CHUNK,), jnp.float32),
        pltpu.VMEM((L,), jnp.float32),
        pltpu.VMEM((L,), jnp.float32),
        pltpu.SemaphoreType.DMA,
        pltpu.SemaphoreType.DMA,
        pltpu.SemaphoreType.DMA,
        pltpu.SemaphoreType.DMA,
    ],
)(_sc_body)


def kernel(rpn_match, rpn_class_logits):
    m = rpn_match.reshape(TOTAL)
    lg = (rpn_class_logits
          .reshape(B, A // GROUP, GROUP, 2)
          .transpose(0, 1, 3, 2)
          .reshape(TOTAL * 2))
    sums, cnts = _rpn_loss_sc(m, lg)
    s = jnp.sum(sums)
    c = jnp.sum(cnts)
    return jnp.where(c > 0, s / jnp.maximum(c, 1.0), jnp.float32(0.0))
